# Initial kernel scaffold; baseline (speedup 1.0000x reference)
import jax
import jax.numpy as jnp
from jax.experimental import pallas as pl


def kernel(x, edge_attr, W_rel, W_root, b_conv, W_lin, b_lin, edge_index, batch):
    raise NotImplementedError("write your pallas kernel here")


# speedup vs baseline: 7.3140x; 7.3140x over previous
import functools

import jax
import jax.numpy as jnp
from jax import lax
from jax.experimental import pallas as pl
from jax.experimental.pallas import tpu as pltpu
from jax.experimental.pallas import tpu_sc as plsc

N = 10000
E = 320000
D = 128
R = 4
G = 16
Np = 10240
CH = 32
NT = 16
WIN = 128
GW = 32
NW = 160
NSEG = NW // GW
EPT = NW * WIN
EP = NT * EPT
RNp = R * Np
RPT = RNp // NT
BLK = 512
NBLK = Np // BLK


def _sc_body(with_cnt, hc0, hc1, hc2, hc3, keys3, src3, *rest):
    if with_cnt:
        a0, a1, a2, a3, cnt = rest[:5]
        scratch = rest[5:]
    else:
        a0, a1, a2, a3 = rest[:4]
        cnt = None
        scratch = rest[4:]
    acc, cntacc, keys_v, src_v, rows_v, zb, zr, ones_v, gsem = scratch

    core = lax.axis_index("c")
    sid = lax.axis_index("s")

    for j in range(WIN):
        zb[j, pl.ds(0, 16)] = jnp.zeros((16,), jnp.float32)
        zb[j, pl.ds(16, 16)] = jnp.zeros((16,), jnp.float32)
    for j in range(WIN // 16):
        zr[pl.ds(j * 16, 16)] = jnp.zeros((16,), jnp.float32)
        ones_v[pl.ds(j * 16, 16)] = jnp.ones((16,), jnp.float32)

    def zero_acc(also_cnt):
        base = sid * RPT
        for w in range(RPT // WIN):
            pltpu.sync_copy(zb, acc.at[pl.ds(base + w * WIN, WIN)])
            if also_cnt:
                pltpu.sync_copy(zr, cntacc.at[pl.ds(base + w * WIN, WIN)])
        plsc.subcore_barrier()

    def do_pass(table, a_out, count):
        def sbody(s, carry):
            pltpu.sync_copy(keys3.at[sid, pl.ds(s * GW, GW)], keys_v)
            pltpu.sync_copy(src3.at[sid, pl.ds(s * GW, GW)], src_v)

            def wbody(j, c2):
                pltpu.async_copy(table.at[src_v.at[j]], rows_v, gsem).wait()
                pltpu.sync_copy(rows_v, acc.at[keys_v.at[j]], add=True)
                if count:
                    pltpu.sync_copy(ones_v, cntacc.at[keys_v.at[j]], add=True)
                return c2
            lax.fori_loop(0, GW, wbody, 0)
            return carry
        lax.fori_loop(0, NSEG, sbody, 0)
        plsc.subcore_barrier()
        base = sid * RPT
        for w in range(RPT // WIN):
            off = base + w * WIN
            pltpu.sync_copy(acc.at[pl.ds(off, WIN)], a_out.at[pl.ds(off, WIN)])
            if count:
                pltpu.sync_copy(cntacc.at[pl.ds(off, WIN)], cnt.at[pl.ds(off, WIN)])
        plsc.subcore_barrier()

    @pl.when(core == 0)
    def _():
        zero_acc(with_cnt)
        do_pass(hc0, a0, with_cnt)
        zero_acc(False)
        do_pass(hc1, a1, False)

    @pl.when(core == 1)
    def _():
        zero_acc(False)
        do_pass(hc2, a2, False)
        zero_acc(False)
        do_pass(hc3, a3, False)


def _make_sc(with_cnt):
    out_type = [jax.ShapeDtypeStruct((RNp, CH), jnp.float32) for _ in range(4)]
    if with_cnt:
        out_type.append(jax.ShapeDtypeStruct((RNp,), jnp.float32))
    return pl.kernel(
        functools.partial(_sc_body, with_cnt),
        out_type=tuple(out_type),
        mesh=plsc.VectorSubcoreMesh(core_axis_name="c", subcore_axis_name="s"),
        scratch_types=(
            pltpu.VMEM_SHARED((RNp, CH), jnp.float32),
            pltpu.VMEM_SHARED((RNp,), jnp.float32),
            pltpu.VMEM((GW, WIN), jnp.int32),
            pltpu.VMEM((GW, WIN), jnp.int32),
            pltpu.VMEM((WIN, CH), jnp.float32),
            pltpu.VMEM((WIN, CH), jnp.float32),
            pltpu.VMEM((WIN,), jnp.float32),
            pltpu.VMEM((WIN,), jnp.float32),
            pltpu.SemaphoreType.DMA,
        ),
        compiler_params=pltpu.CompilerParams(use_tc_tiling_on_sc=False),
    )


_sc_first = _make_sc(True)
_sc_rest = _make_sc(False)


def _layer_body(hc0, hc1, hc2, hc3, a0, a1, a2, a3, cnt_ref, wroot, wrel,
                b_ref, o0, o1, o2, o3, accs):
    r = pl.program_id(1)
    amat = jnp.concatenate(
        [a0[...], a1[...], a2[...], a3[...]], axis=1)
    inv = 1.0 / jnp.maximum(cnt_ref[...], 1.0)
    sel = (lax.broadcasted_iota(jnp.int32, (BLK, R), 1) == r)
    inv_r = jnp.sum(jnp.where(sel, inv, 0.0), axis=1, keepdims=True)
    contrib = lax.dot(amat, wrel[0], preferred_element_type=jnp.float32)
    contrib = contrib * inv_r

    @pl.when(r == 0)
    def _():
        h = jnp.concatenate(
            [hc0[...], hc1[...], hc2[...], hc3[...]], axis=1)
        root = lax.dot(h, wroot[...], preferred_element_type=jnp.float32)
        accs[...] = root + b_ref[...] + contrib

    @pl.when(r > 0)
    def _():
        accs[...] = accs[...] + contrib

    @pl.when(r == R - 1)
    def _():
        z = jnp.maximum(accs[...], 0.0)
        o0[...] = z[:, 0:32]
        o1[...] = z[:, 32:64]
        o2[...] = z[:, 64:96]
        o3[...] = z[:, 96:128]


def _tc_layer(hc, a_chunks, cnt_t, w_root, w_rel, b):
    hc_spec = pl.BlockSpec((BLK, CH), lambda i, r: (i, 0))
    a_spec = pl.BlockSpec((BLK, CH), lambda i, r: (r * NBLK + i, 0))
    return pl.pallas_call(
        _layer_body,
        grid=(NBLK, R),
        in_specs=[hc_spec] * 4 + [a_spec] * 4 + [
            pl.BlockSpec((BLK, R), lambda i, r: (i, 0)),
            pl.BlockSpec((D, D), lambda i, r: (0, 0)),
            pl.BlockSpec((1, D, D), lambda i, r: (r, 0, 0)),
            pl.BlockSpec((1, D), lambda i, r: (0, 0)),
        ],
        out_specs=[pl.BlockSpec((BLK, CH), lambda i, r: (i, 0))] * 4,
        out_shape=[jax.ShapeDtypeStruct((Np, CH), jnp.float32)] * 4,
        scratch_shapes=[pltpu.VMEM((BLK, D), jnp.float32)],
    )(*hc, *a_chunks, cnt_t, w_root, w_rel, b.reshape(1, D))


def _pool_body(hc0, hc1, hc2, hc3, batch_ref, wlin, blin, out_ref, pacc, cacc):
    i = pl.program_id(0)

    @pl.when(i == 0)
    def _():
        pacc[...] = jnp.zeros_like(pacc)
        cacc[...] = jnp.zeros_like(cacc)

    h = jnp.concatenate(
        [hc0[...], hc1[...], hc2[...], hc3[...]], axis=1)
    bt = batch_ref[...]
    m = (bt == lax.broadcasted_iota(jnp.int32, (BLK, G), 1))
    m = m.astype(jnp.float32)
    pacc[...] = pacc[...] + lax.dot_general(
        m, h, (((0,), (0,)), ((), ())), preferred_element_type=jnp.float32)
    cacc[...] = cacc[...] + jnp.sum(m, axis=0)[:, None]

    @pl.when(i == NBLK - 1)
    def _():
        pooled = pacc[...] / jnp.maximum(cacc[...], 1.0)
        out_ref[...] = lax.dot(
            pooled, wlin[...], preferred_element_type=jnp.float32) + blin[...]


def _tc_pool(hc, batch2, w_lin, b_lin):
    hc_spec = pl.BlockSpec((BLK, CH), lambda i: (i, 0))
    return pl.pallas_call(
        _pool_body,
        grid=(NBLK,),
        in_specs=[hc_spec] * 4 + [
            pl.BlockSpec((BLK, 1), lambda i: (i, 0)),
            pl.BlockSpec((D, D), lambda i: (0, 0)),
            pl.BlockSpec((1, D), lambda i: (0, 0)),
        ],
        out_specs=pl.BlockSpec((G, D), lambda i: (0, 0)),
        out_shape=jax.ShapeDtypeStruct((G, D), jnp.float32),
        scratch_shapes=[pltpu.VMEM((G, D), jnp.float32),
                        pltpu.VMEM((G, D), jnp.float32)],
    )(*hc, batch2, w_lin, b_lin.reshape(1, D))


def kernel(x, edge_attr, W_rel, W_root, b_conv, W_lin, b_lin, edge_index, batch):
    src = edge_index[0].astype(jnp.int32)
    dst = edge_index[1].astype(jnp.int32)
    edge_type = jnp.argmax(edge_attr, axis=1).astype(jnp.int32)

    keys = edge_type * Np + dst
    pad = EP - E
    ar = jnp.arange(pad, dtype=jnp.int32)
    keys = jnp.concatenate([keys, (ar % R) * Np + N + (ar % (Np - N))])
    srcp = jnp.concatenate([src, ar % N])
    keys3 = keys.reshape(NT, NW, WIN)
    src3 = srcp.reshape(NT, NW, WIN)

    xp = jnp.concatenate([x, jnp.zeros((Np - N, D), x.dtype)], axis=0)
    hc = tuple(xp[:, c * CH:(c + 1) * CH] for c in range(4))

    batch2 = jnp.concatenate(
        [batch.astype(jnp.int32), jnp.full((Np - N,), G, jnp.int32)]
    ).reshape(Np, 1)

    cnt_t = None
    for l in range(W_rel.shape[0]):
        if l == 0:
            *a_chunks, cnt = _sc_first(*hc, keys3, src3)
            cnt_t = cnt.reshape(R, Np).T
        else:
            a_chunks = _sc_rest(*hc, keys3, src3)
        hc = _tc_layer(hc, a_chunks, cnt_t, W_root[l], W_rel[l], b_conv[l])

    return _tc_pool(hc, batch2, W_lin, b_lin)


# speedup vs baseline: 9.9618x; 1.3620x over previous
import functools

import jax
import jax.numpy as jnp
from jax import lax
from jax.experimental import pallas as pl
from jax.experimental.pallas import tpu as pltpu
from jax.experimental.pallas import tpu_sc as plsc

N = 10000
E = 320000
D = 128
R = 4
G = 16
Np = 10240
CH = 32
NT = 16
WIN = 128
GW = 32
NW = 160
NSEG = NW // GW
EPT = NW * WIN
EP = NT * EPT
RNp = R * Np
RPT = RNp // NT
BLK = 512
NBLK = Np // BLK


def _sc_body(with_cnt, hc0, hc1, hc2, hc3, keys3, src3, *rest):
    if with_cnt:
        a0, a1, a2, a3, cnt = rest[:5]
        scratch = rest[5:]
    else:
        a0, a1, a2, a3 = rest[:4]
        cnt = None
        scratch = rest[4:]
    acc, cntacc, keys_v, src_v, rows0, rows1, zb, zr, ones_v, g0, g1 = scratch

    core = lax.axis_index("c")
    sid = lax.axis_index("s")

    for j in range(WIN):
        zb[j, pl.ds(0, 16)] = jnp.zeros((16,), jnp.float32)
        zb[j, pl.ds(16, 16)] = jnp.zeros((16,), jnp.float32)
    for j in range(WIN // 16):
        zr[pl.ds(j * 16, 16)] = jnp.zeros((16,), jnp.float32)
        ones_v[pl.ds(j * 16, 16)] = jnp.ones((16,), jnp.float32)

    def zero_acc(also_cnt):
        base = sid * RPT
        for w in range(RPT // WIN):
            pltpu.sync_copy(zb, acc.at[pl.ds(base + w * WIN, WIN)])
            if also_cnt:
                pltpu.sync_copy(zr, cntacc.at[pl.ds(base + w * WIN, WIN)])
        plsc.subcore_barrier()

    def do_pass(table, a_out, count):
        def sbody(s, carry):
            pltpu.sync_copy(keys3.at[sid, pl.ds(s * GW, GW)], keys_v)
            pltpu.sync_copy(src3.at[sid, pl.ds(s * GW, GW)], src_v)
            pltpu.async_copy(table.at[src_v.at[0]], rows0, g0)
            pltpu.async_copy(table.at[src_v.at[1]], rows1, g1)

            def pbody(t, c2):
                for buf, sem, w in ((rows0, g0, 2 * t), (rows1, g1, 2 * t + 1)):
                    pltpu.make_async_copy(table.at[src_v.at[0]], buf, sem).wait()
                    pltpu.sync_copy(buf, acc.at[keys_v.at[w]], add=True)
                    if count:
                        pltpu.sync_copy(ones_v, cntacc.at[keys_v.at[w]], add=True)

                    @pl.when(t + 1 < GW // 2)
                    def _():
                        pltpu.async_copy(table.at[src_v.at[w + 2]], buf, sem)
                return c2
            lax.fori_loop(0, GW // 2, pbody, 0)
            return carry
        lax.fori_loop(0, NSEG, sbody, 0)
        plsc.subcore_barrier()
        base = sid * RPT
        for w in range(RPT // WIN):
            off = base + w * WIN
            pltpu.sync_copy(acc.at[pl.ds(off, WIN)], a_out.at[pl.ds(off, WIN)])
            if count:
                pltpu.sync_copy(cntacc.at[pl.ds(off, WIN)], cnt.at[pl.ds(off, WIN)])
        plsc.subcore_barrier()

    @pl.when(core == 0)
    def _():
        zero_acc(with_cnt)
        do_pass(hc0, a0, with_cnt)
        zero_acc(False)
        do_pass(hc1, a1, False)

    @pl.when(core == 1)
    def _():
        zero_acc(False)
        do_pass(hc2, a2, False)
        zero_acc(False)
        do_pass(hc3, a3, False)


def _make_sc(with_cnt):
    out_type = [jax.ShapeDtypeStruct((RNp, CH), jnp.float32) for _ in range(4)]
    if with_cnt:
        out_type.append(jax.ShapeDtypeStruct((RNp,), jnp.float32))
    return pl.kernel(
        functools.partial(_sc_body, with_cnt),
        out_type=tuple(out_type),
        mesh=plsc.VectorSubcoreMesh(core_axis_name="c", subcore_axis_name="s"),
        scratch_types=(
            pltpu.VMEM_SHARED((RNp, CH), jnp.float32),
            pltpu.VMEM_SHARED((RNp,), jnp.float32),
            pltpu.VMEM((GW, WIN), jnp.int32),
            pltpu.VMEM((GW, WIN), jnp.int32),
            pltpu.VMEM((WIN, CH), jnp.float32),
            pltpu.VMEM((WIN, CH), jnp.float32),
            pltpu.VMEM((WIN, CH), jnp.float32),
            pltpu.VMEM((WIN,), jnp.float32),
            pltpu.VMEM((WIN,), jnp.float32),
            pltpu.SemaphoreType.DMA,
            pltpu.SemaphoreType.DMA,
        ),
        compiler_params=pltpu.CompilerParams(use_tc_tiling_on_sc=False),
    )


_sc_first = _make_sc(True)
_sc_rest = _make_sc(False)


def _layer_body(hc0, hc1, hc2, hc3, a0, a1, a2, a3, cnt_ref, wroot, wrel,
                b_ref, o0, o1, o2, o3, accs):
    r = pl.program_id(1)
    amat = jnp.concatenate(
        [a0[...], a1[...], a2[...], a3[...]], axis=1)
    inv = 1.0 / jnp.maximum(cnt_ref[...], 1.0)
    sel = (lax.broadcasted_iota(jnp.int32, (BLK, R), 1) == r)
    inv_r = jnp.sum(jnp.where(sel, inv, 0.0), axis=1, keepdims=True)
    contrib = lax.dot(amat, wrel[0], preferred_element_type=jnp.float32)
    contrib = contrib * inv_r

    @pl.when(r == 0)
    def _():
        h = jnp.concatenate(
            [hc0[...], hc1[...], hc2[...], hc3[...]], axis=1)
        root = lax.dot(h, wroot[...], preferred_element_type=jnp.float32)
        accs[...] = root + b_ref[...] + contrib

    @pl.when(r > 0)
    def _():
        accs[...] = accs[...] + contrib

    @pl.when(r == R - 1)
    def _():
        z = jnp.maximum(accs[...], 0.0)
        o0[...] = z[:, 0:32]
        o1[...] = z[:, 32:64]
        o2[...] = z[:, 64:96]
        o3[...] = z[:, 96:128]


def _tc_layer(hc, a_chunks, cnt_t, w_root, w_rel, b):
    hc_spec = pl.BlockSpec((BLK, CH), lambda i, r: (i, 0))
    a_spec = pl.BlockSpec((BLK, CH), lambda i, r: (r * NBLK + i, 0))
    return pl.pallas_call(
        _layer_body,
        grid=(NBLK, R),
        in_specs=[hc_spec] * 4 + [a_spec] * 4 + [
            pl.BlockSpec((BLK, R), lambda i, r: (i, 0)),
            pl.BlockSpec((D, D), lambda i, r: (0, 0)),
            pl.BlockSpec((1, D, D), lambda i, r: (r, 0, 0)),
            pl.BlockSpec((1, D), lambda i, r: (0, 0)),
        ],
        out_specs=[pl.BlockSpec((BLK, CH), lambda i, r: (i, 0))] * 4,
        out_shape=[jax.ShapeDtypeStruct((Np, CH), jnp.float32)] * 4,
        scratch_shapes=[pltpu.VMEM((BLK, D), jnp.float32)],
    )(*hc, *a_chunks, cnt_t, w_root, w_rel, b.reshape(1, D))


def _pool_body(hc0, hc1, hc2, hc3, batch_ref, wlin, blin, out_ref, pacc, cacc):
    i = pl.program_id(0)

    @pl.when(i == 0)
    def _():
        pacc[...] = jnp.zeros_like(pacc)
        cacc[...] = jnp.zeros_like(cacc)

    h = jnp.concatenate(
        [hc0[...], hc1[...], hc2[...], hc3[...]], axis=1)
    bt = batch_ref[...]
    m = (bt == lax.broadcasted_iota(jnp.int32, (BLK, G), 1))
    m = m.astype(jnp.float32)
    pacc[...] = pacc[...] + lax.dot_general(
        m, h, (((0,), (0,)), ((), ())), preferred_element_type=jnp.float32)
    cacc[...] = cacc[...] + jnp.sum(m, axis=0)[:, None]

    @pl.when(i == NBLK - 1)
    def _():
        pooled = pacc[...] / jnp.maximum(cacc[...], 1.0)
        out_ref[...] = lax.dot(
            pooled, wlin[...], preferred_element_type=jnp.float32) + blin[...]


def _tc_pool(hc, batch2, w_lin, b_lin):
    hc_spec = pl.BlockSpec((BLK, CH), lambda i: (i, 0))
    return pl.pallas_call(
        _pool_body,
        grid=(NBLK,),
        in_specs=[hc_spec] * 4 + [
            pl.BlockSpec((BLK, 1), lambda i: (i, 0)),
            pl.BlockSpec((D, D), lambda i: (0, 0)),
            pl.BlockSpec((1, D), lambda i: (0, 0)),
        ],
        out_specs=pl.BlockSpec((G, D), lambda i: (0, 0)),
        out_shape=jax.ShapeDtypeStruct((G, D), jnp.float32),
        scratch_shapes=[pltpu.VMEM((G, D), jnp.float32),
                        pltpu.VMEM((G, D), jnp.float32)],
    )(*hc, batch2, w_lin, b_lin.reshape(1, D))


def kernel(x, edge_attr, W_rel, W_root, b_conv, W_lin, b_lin, edge_index, batch):
    src = edge_index[0].astype(jnp.int32)
    dst = edge_index[1].astype(jnp.int32)
    edge_type = jnp.argmax(edge_attr, axis=1).astype(jnp.int32)

    keys = edge_type * Np + dst
    pad = EP - E
    ar = jnp.arange(pad, dtype=jnp.int32)
    keys = jnp.concatenate([keys, (ar % R) * Np + N + (ar % (Np - N))])
    srcp = jnp.concatenate([src, ar % N])
    keys3 = keys.reshape(NT, NW, WIN)
    src3 = srcp.reshape(NT, NW, WIN)

    xp = jnp.concatenate([x, jnp.zeros((Np - N, D), x.dtype)], axis=0)
    hc = tuple(xp[:, c * CH:(c + 1) * CH] for c in range(4))

    batch2 = jnp.concatenate(
        [batch.astype(jnp.int32), jnp.full((Np - N,), G, jnp.int32)]
    ).reshape(Np, 1)

    cnt_t = None
    for l in range(W_rel.shape[0]):
        if l == 0:
            *a_chunks, cnt = _sc_first(*hc, keys3, src3)
            cnt_t = cnt.reshape(R, Np).T
        else:
            a_chunks = _sc_rest(*hc, keys3, src3)
        hc = _tc_layer(hc, a_chunks, cnt_t, W_root[l], W_rel[l], b_conv[l])

    return _tc_pool(hc, batch2, W_lin, b_lin)


# speedup vs baseline: 11.0327x; 1.1075x over previous
import functools

import jax
import jax.numpy as jnp
from jax import lax
from jax.experimental import pallas as pl
from jax.experimental.pallas import tpu as pltpu
from jax.experimental.pallas import tpu_sc as plsc

N = 10000
E = 320000
D = 128
R = 4
G = 16
Np = 10240
CH = 32
NT = 16
WIN = 128
GW = 32
NW = 160
NSEG = NW // GW
EPT = NW * WIN
EP = NT * EPT
RNp = R * Np
RPT = RNp // NT
BLK = 1024
NBLK = Np // BLK


def _sc_body(with_cnt, hc0, hc1, hc2, hc3, keys3, src3, *rest):
    if with_cnt:
        a0, a1, a2, a3, cnt = rest[:5]
        scratch = rest[5:]
    else:
        a0, a1, a2, a3 = rest[:4]
        cnt = None
        scratch = rest[4:]
    acc, cntacc, keys_v, src_v, rows0, rows1, zb, zr, ones_v, g0, g1 = scratch

    core = lax.axis_index("c")
    sid = lax.axis_index("s")

    for j in range(WIN):
        zb[j, pl.ds(0, 16)] = jnp.zeros((16,), jnp.float32)
        zb[j, pl.ds(16, 16)] = jnp.zeros((16,), jnp.float32)
    for j in range(WIN // 16):
        zr[pl.ds(j * 16, 16)] = jnp.zeros((16,), jnp.float32)
        ones_v[pl.ds(j * 16, 16)] = jnp.ones((16,), jnp.float32)

    def zero_acc(also_cnt):
        base = sid * RPT
        for w in range(RPT // WIN):
            pltpu.sync_copy(zb, acc.at[pl.ds(base + w * WIN, WIN)])
            if also_cnt:
                pltpu.sync_copy(zr, cntacc.at[pl.ds(base + w * WIN, WIN)])
        plsc.subcore_barrier()

    def do_pass(table, a_out, count):
        def sbody(s, carry):
            pltpu.sync_copy(keys3.at[sid, pl.ds(s * GW, GW)], keys_v)
            pltpu.sync_copy(src3.at[sid, pl.ds(s * GW, GW)], src_v)
            pltpu.async_copy(table.at[src_v.at[0]], rows0, g0)
            pltpu.async_copy(table.at[src_v.at[1]], rows1, g1)

            def pbody(t, c2):
                for buf, sem, w in ((rows0, g0, 2 * t), (rows1, g1, 2 * t + 1)):
                    pltpu.make_async_copy(table.at[src_v.at[0]], buf, sem).wait()
                    pltpu.sync_copy(buf, acc.at[keys_v.at[w]], add=True)
                    if count:
                        pltpu.sync_copy(ones_v, cntacc.at[keys_v.at[w]], add=True)

                    @pl.when(t + 1 < GW // 2)
                    def _():
                        pltpu.async_copy(table.at[src_v.at[w + 2]], buf, sem)
                return c2
            lax.fori_loop(0, GW // 2, pbody, 0)
            return carry
        lax.fori_loop(0, NSEG, sbody, 0)
        plsc.subcore_barrier()
        base = sid * RPT
        for w in range(RPT // WIN):
            off = base + w * WIN
            pltpu.sync_copy(acc.at[pl.ds(off, WIN)], a_out.at[pl.ds(off, WIN)])
            if count:
                pltpu.sync_copy(cntacc.at[pl.ds(off, WIN)], cnt.at[pl.ds(off, WIN)])
        plsc.subcore_barrier()

    @pl.when(core == 0)
    def _():
        zero_acc(with_cnt)
        do_pass(hc0, a0, with_cnt)
        zero_acc(False)
        do_pass(hc1, a1, False)

    @pl.when(core == 1)
    def _():
        zero_acc(False)
        do_pass(hc2, a2, False)
        zero_acc(False)
        do_pass(hc3, a3, False)


def _make_sc(with_cnt):
    out_type = [jax.ShapeDtypeStruct((RNp, CH), jnp.float32) for _ in range(4)]
    if with_cnt:
        out_type.append(jax.ShapeDtypeStruct((RNp,), jnp.float32))
    return pl.kernel(
        functools.partial(_sc_body, with_cnt),
        out_type=tuple(out_type),
        mesh=plsc.VectorSubcoreMesh(core_axis_name="c", subcore_axis_name="s"),
        scratch_types=(
            pltpu.VMEM_SHARED((RNp, CH), jnp.float32),
            pltpu.VMEM_SHARED((RNp,), jnp.float32),
            pltpu.VMEM((GW, WIN), jnp.int32),
            pltpu.VMEM((GW, WIN), jnp.int32),
            pltpu.VMEM((WIN, CH), jnp.float32),
            pltpu.VMEM((WIN, CH), jnp.float32),
            pltpu.VMEM((WIN, CH), jnp.float32),
            pltpu.VMEM((WIN,), jnp.float32),
            pltpu.VMEM((WIN,), jnp.float32),
            pltpu.SemaphoreType.DMA,
            pltpu.SemaphoreType.DMA,
        ),
        compiler_params=pltpu.CompilerParams(use_tc_tiling_on_sc=False),
    )


_sc_first = _make_sc(True)
_sc_rest = _make_sc(False)


def _layer_body(hc0, hc1, hc2, hc3, *refs):
    a = refs[:16]
    cnt_ref, wroot, wrel, b_ref, o0, o1, o2, o3 = refs[16:]
    h = jnp.concatenate(
        [hc0[...], hc1[...], hc2[...], hc3[...]], axis=1)
    acc = lax.dot(h, wroot[...], preferred_element_type=jnp.float32)
    acc = acc + b_ref[...]
    inv = 1.0 / jnp.maximum(cnt_ref[...], 1.0)
    for r in range(R):
        amat = jnp.concatenate(
            [a[4 * r + c][...] for c in range(4)], axis=1)
        amat = amat * inv[:, r:r + 1]
        acc = acc + lax.dot(amat, wrel[r],
                            preferred_element_type=jnp.float32)
    z = jnp.maximum(acc, 0.0)
    o0[...] = z[:, 0:32]
    o1[...] = z[:, 32:64]
    o2[...] = z[:, 64:96]
    o3[...] = z[:, 96:128]


def _a_spec(r):
    return pl.BlockSpec((BLK, CH), lambda i, r=r: (r * NBLK + i, 0))


def _tc_layer(hc, a_chunks, cnt_t, w_root, w_rel, b):
    hc_spec = pl.BlockSpec((BLK, CH), lambda i: (i, 0))
    a_args = [a_chunks[c] for r in range(R) for c in range(4)]
    a_specs = [_a_spec(r) for r in range(R) for c in range(4)]
    return pl.pallas_call(
        _layer_body,
        grid=(NBLK,),
        in_specs=[hc_spec] * 4 + a_specs + [
            pl.BlockSpec((BLK, R), lambda i: (i, 0)),
            pl.BlockSpec((D, D), lambda i: (0, 0)),
            pl.BlockSpec((R, D, D), lambda i: (0, 0, 0)),
            pl.BlockSpec((1, D), lambda i: (0, 0)),
        ],
        out_specs=[pl.BlockSpec((BLK, CH), lambda i: (i, 0))] * 4,
        out_shape=[jax.ShapeDtypeStruct((Np, CH), jnp.float32)] * 4,
    )(*hc, *a_args, cnt_t, w_root, w_rel, b.reshape(1, D))


def _pool_body(hc0, hc1, hc2, hc3, batch_ref, wlin, blin, out_ref, pacc, cacc):
    i = pl.program_id(0)

    @pl.when(i == 0)
    def _():
        pacc[...] = jnp.zeros_like(pacc)
        cacc[...] = jnp.zeros_like(cacc)

    h = jnp.concatenate(
        [hc0[...], hc1[...], hc2[...], hc3[...]], axis=1)
    bt = batch_ref[...]
    m = (bt == lax.broadcasted_iota(jnp.int32, (BLK, G), 1))
    m = m.astype(jnp.float32)
    pacc[...] = pacc[...] + lax.dot_general(
        m, h, (((0,), (0,)), ((), ())), preferred_element_type=jnp.float32)
    cacc[...] = cacc[...] + jnp.sum(m, axis=0)[:, None]

    @pl.when(i == NBLK - 1)
    def _():
        pooled = pacc[...] / jnp.maximum(cacc[...], 1.0)
        out_ref[...] = lax.dot(
            pooled, wlin[...], preferred_element_type=jnp.float32) + blin[...]


def _tc_pool(hc, batch2, w_lin, b_lin):
    hc_spec = pl.BlockSpec((BLK, CH), lambda i: (i, 0))
    return pl.pallas_call(
        _pool_body,
        grid=(NBLK,),
        in_specs=[hc_spec] * 4 + [
            pl.BlockSpec((BLK, 1), lambda i: (i, 0)),
            pl.BlockSpec((D, D), lambda i: (0, 0)),
            pl.BlockSpec((1, D), lambda i: (0, 0)),
        ],
        out_specs=pl.BlockSpec((G, D), lambda i: (0, 0)),
        out_shape=jax.ShapeDtypeStruct((G, D), jnp.float32),
        scratch_shapes=[pltpu.VMEM((G, D), jnp.float32),
                        pltpu.VMEM((G, D), jnp.float32)],
    )(*hc, batch2, w_lin, b_lin.reshape(1, D))


def kernel(x, edge_attr, W_rel, W_root, b_conv, W_lin, b_lin, edge_index, batch):
    src = edge_index[0].astype(jnp.int32)
    dst = edge_index[1].astype(jnp.int32)
    edge_type = jnp.argmax(edge_attr, axis=1).astype(jnp.int32)

    keys = edge_type * Np + dst
    pad = EP - E
    ar = jnp.arange(pad, dtype=jnp.int32)
    keys = jnp.concatenate([keys, (ar % R) * Np + N + (ar % (Np - N))])
    srcp = jnp.concatenate([src, ar % N])
    keys3 = keys.reshape(NT, NW, WIN)
    src3 = srcp.reshape(NT, NW, WIN)

    xp = jnp.concatenate([x, jnp.zeros((Np - N, D), x.dtype)], axis=0)
    hc = tuple(xp[:, c * CH:(c + 1) * CH] for c in range(4))

    batch2 = jnp.concatenate(
        [batch.astype(jnp.int32), jnp.full((Np - N,), G, jnp.int32)]
    ).reshape(Np, 1)

    cnt_t = None
    for l in range(W_rel.shape[0]):
        if l == 0:
            *a_chunks, cnt = _sc_first(*hc, keys3, src3)
            cnt_t = cnt.reshape(R, Np).T
        else:
            a_chunks = _sc_rest(*hc, keys3, src3)
        hc = _tc_layer(hc, a_chunks, cnt_t, W_root[l], W_rel[l], b_conv[l])

    return _tc_pool(hc, batch2, W_lin, b_lin)


# speedup vs baseline: 11.8077x; 1.0702x over previous
import functools

import jax
import jax.numpy as jnp
from jax import lax
from jax.experimental import pallas as pl
from jax.experimental.pallas import tpu as pltpu
from jax.experimental.pallas import tpu_sc as plsc

N = 10000
E = 320000
D = 128
R = 4
G = 16
Np = 10240
CH = 32
NT = 16
WIN = 128
GW = 32
NW = 160
NSEG = NW // GW
EPT = NW * WIN
EP = NT * EPT
RNp = R * Np
RPT = RNp // NT
BLK = 1024
NBLK = Np // BLK


def _sc_body(with_cnt, hc0, hc1, hc2, hc3, keys3, src3, *rest):
    if with_cnt:
        a0, a1, a2, a3, cnt = rest[:5]
        scratch = rest[5:]
    else:
        a0, a1, a2, a3 = rest[:4]
        cnt = None
        scratch = rest[4:]
    (acc, cntacc, keys_v, src_v, rows0, rows1, rows2, rows3,
     zb, zr, ones_v, g0, g1, g2, g3, s0, s1, s2, s3) = scratch
    rows = (rows0, rows1, rows2, rows3)
    gsem = (g0, g1, g2, g3)
    ssem = (s0, s1, s2, s3)

    core = lax.axis_index("c")
    sid = lax.axis_index("s")

    for j in range(WIN):
        zb[j, pl.ds(0, 16)] = jnp.zeros((16,), jnp.float32)
        zb[j, pl.ds(16, 16)] = jnp.zeros((16,), jnp.float32)
    for j in range(WIN // 16):
        zr[pl.ds(j * 16, 16)] = jnp.zeros((16,), jnp.float32)
        ones_v[pl.ds(j * 16, 16)] = jnp.ones((16,), jnp.float32)

    def zero_acc(also_cnt):
        base = sid * RPT
        for w in range(RPT // WIN):
            pltpu.sync_copy(zb, acc.at[pl.ds(base + w * WIN, WIN)])
            if also_cnt:
                pltpu.sync_copy(zr, cntacc.at[pl.ds(base + w * WIN, WIN)])
        plsc.subcore_barrier()

    def do_pass(table, a_out, count):
        def wait_dma(buf, sem):
            pltpu.make_async_copy(table.at[src_v.at[0]], buf, sem).wait()

        def sbody(s, carry):
            pltpu.sync_copy(keys3.at[sid, pl.ds(s * GW, GW)], keys_v)
            pltpu.sync_copy(src3.at[sid, pl.ds(s * GW, GW)], src_v)
            pltpu.async_copy(table.at[src_v.at[0]], rows0, g0)
            pltpu.async_copy(table.at[src_v.at[1]], rows1, g1)

            def qbody(t, c2):
                for k in range(4):
                    w = 4 * t + k
                    b = k
                    b2 = (k + 2) % 4
                    wait_dma(rows[b], gsem[b])
                    pltpu.async_copy(rows[b], acc.at[keys_v.at[w]],
                                     ssem[b], add=True)
                    if count:
                        pltpu.sync_copy(ones_v, cntacc.at[keys_v.at[w]],
                                        add=True)
                    if k < 2:
                        @pl.when(t > 0)
                        def _():
                            wait_dma(rows[b2], ssem[b2])
                        pltpu.async_copy(table.at[src_v.at[w + 2]],
                                         rows[b2], gsem[b2])
                    else:
                        wait_dma(rows[b2], ssem[b2])

                        @pl.when(t < GW // 4 - 1)
                        def _():
                            pltpu.async_copy(table.at[src_v.at[w + 2]],
                                             rows[b2], gsem[b2])
                return c2
            lax.fori_loop(0, GW // 4, qbody, 0)
            wait_dma(rows2, s2)
            wait_dma(rows3, s3)
            return carry
        lax.fori_loop(0, NSEG, sbody, 0)
        plsc.subcore_barrier()
        base = sid * RPT
        for w in range(RPT // WIN):
            off = base + w * WIN
            pltpu.sync_copy(acc.at[pl.ds(off, WIN)], a_out.at[pl.ds(off, WIN)])
            if count:
                pltpu.sync_copy(cntacc.at[pl.ds(off, WIN)], cnt.at[pl.ds(off, WIN)])
        plsc.subcore_barrier()

    @pl.when(core == 0)
    def _():
        zero_acc(with_cnt)
        do_pass(hc0, a0, with_cnt)
        zero_acc(False)
        do_pass(hc1, a1, False)

    @pl.when(core == 1)
    def _():
        zero_acc(False)
        do_pass(hc2, a2, False)
        zero_acc(False)
        do_pass(hc3, a3, False)


def _make_sc(with_cnt):
    out_type = [jax.ShapeDtypeStruct((RNp, CH), jnp.float32) for _ in range(4)]
    if with_cnt:
        out_type.append(jax.ShapeDtypeStruct((RNp,), jnp.float32))
    return pl.kernel(
        functools.partial(_sc_body, with_cnt),
        out_type=tuple(out_type),
        mesh=plsc.VectorSubcoreMesh(core_axis_name="c", subcore_axis_name="s"),
        scratch_types=(
            pltpu.VMEM_SHARED((RNp, CH), jnp.float32),
            pltpu.VMEM_SHARED((RNp,), jnp.float32),
            pltpu.VMEM((GW, WIN), jnp.int32),
            pltpu.VMEM((GW, WIN), jnp.int32),
            pltpu.VMEM((WIN, CH), jnp.float32),
            pltpu.VMEM((WIN, CH), jnp.float32),
            pltpu.VMEM((WIN, CH), jnp.float32),
            pltpu.VMEM((WIN, CH), jnp.float32),
            pltpu.VMEM((WIN, CH), jnp.float32),
            pltpu.VMEM((WIN,), jnp.float32),
            pltpu.VMEM((WIN,), jnp.float32),
            pltpu.SemaphoreType.DMA,
            pltpu.SemaphoreType.DMA,
            pltpu.SemaphoreType.DMA,
            pltpu.SemaphoreType.DMA,
            pltpu.SemaphoreType.DMA,
            pltpu.SemaphoreType.DMA,
            pltpu.SemaphoreType.DMA,
            pltpu.SemaphoreType.DMA,
        ),
        compiler_params=pltpu.CompilerParams(use_tc_tiling_on_sc=False),
    )


_sc_first = _make_sc(True)
_sc_rest = _make_sc(False)


def _layer_body(hc0, hc1, hc2, hc3, *refs):
    a = refs[:16]
    cnt_ref, wroot, wrel, b_ref, o0, o1, o2, o3 = refs[16:]
    h = jnp.concatenate(
        [hc0[...], hc1[...], hc2[...], hc3[...]], axis=1)
    acc = lax.dot(h, wroot[...], preferred_element_type=jnp.float32)
    acc = acc + b_ref[...]
    inv = 1.0 / jnp.maximum(cnt_ref[...], 1.0)
    for r in range(R):
        amat = jnp.concatenate(
            [a[4 * r + c][...] for c in range(4)], axis=1)
        amat = amat * inv[:, r:r + 1]
        acc = acc + lax.dot(amat, wrel[r],
                            preferred_element_type=jnp.float32)
    z = jnp.maximum(acc, 0.0)
    o0[...] = z[:, 0:32]
    o1[...] = z[:, 32:64]
    o2[...] = z[:, 64:96]
    o3[...] = z[:, 96:128]


def _a_spec(r):
    return pl.BlockSpec((BLK, CH), lambda i, r=r: (r * NBLK + i, 0))


def _tc_layer(hc, a_chunks, cnt_t, w_root, w_rel, b):
    hc_spec = pl.BlockSpec((BLK, CH), lambda i: (i, 0))
    a_args = [a_chunks[c] for r in range(R) for c in range(4)]
    a_specs = [_a_spec(r) for r in range(R) for c in range(4)]
    return pl.pallas_call(
        _layer_body,
        grid=(NBLK,),
        in_specs=[hc_spec] * 4 + a_specs + [
            pl.BlockSpec((BLK, R), lambda i: (i, 0)),
            pl.BlockSpec((D, D), lambda i: (0, 0)),
            pl.BlockSpec((R, D, D), lambda i: (0, 0, 0)),
            pl.BlockSpec((1, D), lambda i: (0, 0)),
        ],
        out_specs=[pl.BlockSpec((BLK, CH), lambda i: (i, 0))] * 4,
        out_shape=[jax.ShapeDtypeStruct((Np, CH), jnp.float32)] * 4,
    )(*hc, *a_args, cnt_t, w_root, w_rel, b.reshape(1, D))


def _pool_body(hc0, hc1, hc2, hc3, batch_ref, wlin, blin, out_ref, pacc, cacc):
    i = pl.program_id(0)

    @pl.when(i == 0)
    def _():
        pacc[...] = jnp.zeros_like(pacc)
        cacc[...] = jnp.zeros_like(cacc)

    h = jnp.concatenate(
        [hc0[...], hc1[...], hc2[...], hc3[...]], axis=1)
    bt = batch_ref[...]
    m = (bt == lax.broadcasted_iota(jnp.int32, (BLK, G), 1))
    m = m.astype(jnp.float32)
    pacc[...] = pacc[...] + lax.dot_general(
        m, h, (((0,), (0,)), ((), ())), preferred_element_type=jnp.float32)
    cacc[...] = cacc[...] + jnp.sum(m, axis=0)[:, None]

    @pl.when(i == NBLK - 1)
    def _():
        pooled = pacc[...] / jnp.maximum(cacc[...], 1.0)
        out_ref[...] = lax.dot(
            pooled, wlin[...], preferred_element_type=jnp.float32) + blin[...]


def _tc_pool(hc, batch2, w_lin, b_lin):
    hc_spec = pl.BlockSpec((BLK, CH), lambda i: (i, 0))
    return pl.pallas_call(
        _pool_body,
        grid=(NBLK,),
        in_specs=[hc_spec] * 4 + [
            pl.BlockSpec((BLK, 1), lambda i: (i, 0)),
            pl.BlockSpec((D, D), lambda i: (0, 0)),
            pl.BlockSpec((1, D), lambda i: (0, 0)),
        ],
        out_specs=pl.BlockSpec((G, D), lambda i: (0, 0)),
        out_shape=jax.ShapeDtypeStruct((G, D), jnp.float32),
        scratch_shapes=[pltpu.VMEM((G, D), jnp.float32),
                        pltpu.VMEM((G, D), jnp.float32)],
    )(*hc, batch2, w_lin, b_lin.reshape(1, D))


def kernel(x, edge_attr, W_rel, W_root, b_conv, W_lin, b_lin, edge_index, batch):
    src = edge_index[0].astype(jnp.int32)
    dst = edge_index[1].astype(jnp.int32)
    edge_type = jnp.argmax(edge_attr, axis=1).astype(jnp.int32)

    keys = edge_type * Np + dst
    pad = EP - E
    ar = jnp.arange(pad, dtype=jnp.int32)
    keys = jnp.concatenate([keys, (ar % R) * Np + N + (ar % (Np - N))])
    srcp = jnp.concatenate([src, ar % N])
    keys3 = keys.reshape(NT, NW, WIN)
    src3 = srcp.reshape(NT, NW, WIN)

    xp = jnp.concatenate([x, jnp.zeros((Np - N, D), x.dtype)], axis=0)
    hc = tuple(xp[:, c * CH:(c + 1) * CH] for c in range(4))

    batch2 = jnp.concatenate(
        [batch.astype(jnp.int32), jnp.full((Np - N,), G, jnp.int32)]
    ).reshape(Np, 1)

    cnt_t = None
    for l in range(W_rel.shape[0]):
        if l == 0:
            *a_chunks, cnt = _sc_first(*hc, keys3, src3)
            cnt_t = cnt.reshape(R, Np).T
        else:
            a_chunks = _sc_rest(*hc, keys3, src3)
        hc = _tc_layer(hc, a_chunks, cnt_t, W_root[l], W_rel[l], b_conv[l])

    return _tc_pool(hc, batch2, W_lin, b_lin)


# speedup vs baseline: 15.7401x; 1.3330x over previous
import functools

import jax
import jax.numpy as jnp
from jax import lax
from jax.experimental import pallas as pl
from jax.experimental.pallas import tpu as pltpu
from jax.experimental.pallas import tpu_sc as plsc

N = 10000
E = 320000
D = 128
R = 4
G = 16
Np = 10240
CH = 32
NT = 16
WIN = 128
GW = 32
NW = 160
NSEG = NW // GW
EPT = NW * WIN
EP = NT * EPT
RNp = R * Np
RPT = RNp // NT
BLK = 1024
NBLK = Np // BLK


def _sc_body(with_cnt, hc0, hc1, hc2, hc3, keys3, src3, *rest):
    if with_cnt:
        a0, a1, a2, a3, cnt = rest[:5]
        scratch = rest[5:]
    else:
        a0, a1, a2, a3 = rest[:4]
        cnt = None
        scratch = rest[4:]
    (acc, cntacc, keys_v, src_v, rows0, rows1, rows2, rows3,
     zb, zr, ones_v, g0, g1, g2, g3, s0, s1, s2, s3) = scratch
    rows = (rows0, rows1, rows2, rows3)
    gsem = (g0, g1, g2, g3)
    ssem = (s0, s1, s2, s3)

    core = lax.axis_index("c")
    sid = lax.axis_index("s")

    for j in range(WIN):
        zb[j, pl.ds(0, 16)] = jnp.zeros((16,), jnp.float32)
        zb[j, pl.ds(16, 16)] = jnp.zeros((16,), jnp.float32)
    for j in range(WIN // 16):
        zr[pl.ds(j * 16, 16)] = jnp.zeros((16,), jnp.float32)
        ones_v[pl.ds(j * 16, 16)] = jnp.ones((16,), jnp.float32)

    def zero_acc(also_cnt):
        base = sid * RPT
        for w in range(RPT // WIN):
            pltpu.sync_copy(zb, acc.at[pl.ds(base + w * WIN, WIN)])
            if also_cnt:
                pltpu.sync_copy(zr, cntacc.at[pl.ds(base + w * WIN, WIN)])
        plsc.subcore_barrier()

    def do_pass(table, a_out, count):
        def wait_dma(buf, sem):
            pltpu.make_async_copy(table.at[src_v.at[0]], buf, sem).wait()

        def sbody(s, carry):
            pltpu.sync_copy(keys3.at[sid, pl.ds(s * GW, GW)], keys_v)
            pltpu.sync_copy(src3.at[sid, pl.ds(s * GW, GW)], src_v)
            pltpu.async_copy(table.at[src_v.at[0]], rows0, g0)
            pltpu.async_copy(table.at[src_v.at[1]], rows1, g1)

            def qbody(t, c2):
                for k in range(4):
                    w = 4 * t + k
                    b = k
                    b2 = (k + 2) % 4
                    wait_dma(rows[b], gsem[b])
                    pltpu.async_copy(rows[b], acc.at[keys_v.at[w]],
                                     ssem[b], add=True)
                    if count:
                        pltpu.sync_copy(ones_v, cntacc.at[keys_v.at[w]],
                                        add=True)
                    if k < 2:
                        @pl.when(t > 0)
                        def _():
                            wait_dma(rows[b2], ssem[b2])
                        pltpu.async_copy(table.at[src_v.at[w + 2]],
                                         rows[b2], gsem[b2])
                    else:
                        wait_dma(rows[b2], ssem[b2])

                        @pl.when(t < GW // 4 - 1)
                        def _():
                            pltpu.async_copy(table.at[src_v.at[w + 2]],
                                             rows[b2], gsem[b2])
                return c2
            lax.fori_loop(0, GW // 4, qbody, 0)
            wait_dma(rows2, s2)
            wait_dma(rows3, s3)
            return carry
        lax.fori_loop(0, NSEG, sbody, 0)
        plsc.subcore_barrier()
        base = sid * RPT
        for w in range(RPT // WIN):
            off = base + w * WIN
            pltpu.sync_copy(acc.at[pl.ds(off, WIN)], a_out.at[pl.ds(off, WIN)])
            if count:
                pltpu.sync_copy(cntacc.at[pl.ds(off, WIN)], cnt.at[pl.ds(off, WIN)])
        plsc.subcore_barrier()

    @pl.when(core == 0)
    def _():
        zero_acc(with_cnt)
        do_pass(hc0, a0, with_cnt)
        zero_acc(False)
        do_pass(hc1, a1, False)

    @pl.when(core == 1)
    def _():
        zero_acc(False)
        do_pass(hc2, a2, False)
        zero_acc(False)
        do_pass(hc3, a3, False)


def _make_sc(with_cnt):
    out_type = [jax.ShapeDtypeStruct((RNp, CH), jnp.float32) for _ in range(4)]
    if with_cnt:
        out_type.append(jax.ShapeDtypeStruct((RNp,), jnp.float32))
    return pl.kernel(
        functools.partial(_sc_body, with_cnt),
        out_type=tuple(out_type),
        mesh=plsc.VectorSubcoreMesh(core_axis_name="c", subcore_axis_name="s"),
        scratch_types=(
            pltpu.VMEM_SHARED((RNp, CH), jnp.float32),
            pltpu.VMEM_SHARED((RNp,), jnp.float32),
            pltpu.VMEM((GW, WIN), jnp.int32),
            pltpu.VMEM((GW, WIN), jnp.int32),
            pltpu.VMEM((WIN, CH), jnp.float32),
            pltpu.VMEM((WIN, CH), jnp.float32),
            pltpu.VMEM((WIN, CH), jnp.float32),
            pltpu.VMEM((WIN, CH), jnp.float32),
            pltpu.VMEM((WIN, CH), jnp.float32),
            pltpu.VMEM((WIN,), jnp.float32),
            pltpu.VMEM((WIN,), jnp.float32),
            pltpu.SemaphoreType.DMA,
            pltpu.SemaphoreType.DMA,
            pltpu.SemaphoreType.DMA,
            pltpu.SemaphoreType.DMA,
            pltpu.SemaphoreType.DMA,
            pltpu.SemaphoreType.DMA,
            pltpu.SemaphoreType.DMA,
            pltpu.SemaphoreType.DMA,
        ),
        compiler_params=pltpu.CompilerParams(use_tc_tiling_on_sc=False),
    )


_sc_first = _make_sc(True)
_sc_rest = _make_sc(False)


PB = BLK // 4


def _layer_body(hp0, hp1, hp2, hp3, *refs):
    a = refs[:16]
    cntp, wroot, wrel, b_ref, o0, o1, o2, o3 = refs[16:]
    hp = (hp0, hp1, hp2, hp3)
    invp = 1.0 / jnp.maximum(cntp[...], 1.0)
    ys = []
    for k in range(4):
        hk = jnp.concatenate(
            [hp[c][:, 32 * k:32 * k + 32] for c in range(4)], axis=1)
        yk = lax.dot(hk, wroot[...], preferred_element_type=jnp.float32)
        yk = yk + b_ref[...]
        for r in range(R):
            zrk = jnp.concatenate(
                [a[4 * r + c][:, 32 * k:32 * k + 32] for c in range(4)],
                axis=1)
            yk = yk + lax.dot(
                zrk, wrel[r], preferred_element_type=jnp.float32
            ) * invp[:, 4 * k + r:4 * k + r + 1]
        ys.append(jnp.maximum(yk, 0.0))
    outs = (o0, o1, o2, o3)
    for c in range(4):
        outs[c][...] = jnp.concatenate(
            [ys[k][:, 32 * c:32 * c + 32] for k in range(4)], axis=1)


def _a_spec(r):
    return pl.BlockSpec((PB, D), lambda i, r=r: (r * NBLK + i, 0))


def _tc_layer(hp, ap, cntp, w_root, w_rel, b):
    hp_spec = pl.BlockSpec((PB, D), lambda i: (i, 0))
    a_args = [ap[c] for r in range(R) for c in range(4)]
    a_specs = [_a_spec(r) for r in range(R) for c in range(4)]
    return pl.pallas_call(
        _layer_body,
        grid=(NBLK,),
        in_specs=[hp_spec] * 4 + a_specs + [
            pl.BlockSpec((PB, 4 * R), lambda i: (i, 0)),
            pl.BlockSpec((D, D), lambda i: (0, 0)),
            pl.BlockSpec((R, D, D), lambda i: (0, 0, 0)),
            pl.BlockSpec((1, D), lambda i: (0, 0)),
        ],
        out_specs=[pl.BlockSpec((PB, D), lambda i: (i, 0))] * 4,
        out_shape=[jax.ShapeDtypeStruct((Np // 4, D), jnp.float32)] * 4,
    )(*hp, *a_args, cntp, w_root, w_rel, b.reshape(1, D))


def _pool_body(hp0, hp1, hp2, hp3, batch_ref, wlin, blin, out_ref, pacc, cacc):
    i = pl.program_id(0)
    hp = (hp0, hp1, hp2, hp3)

    @pl.when(i == 0)
    def _():
        pacc[...] = jnp.zeros_like(pacc)
        cacc[...] = jnp.zeros_like(cacc)

    bt = batch_ref[...]
    for k in range(4):
        hk = jnp.concatenate(
            [hp[c][:, 32 * k:32 * k + 32] for c in range(4)], axis=1)
        m = (bt[:, k:k + 1] == lax.broadcasted_iota(jnp.int32, (PB, G), 1))
        m = m.astype(jnp.float32)
        pacc[...] = pacc[...] + lax.dot_general(
            m, hk, (((0,), (0,)), ((), ())),
            preferred_element_type=jnp.float32)
        cacc[...] = cacc[...] + jnp.sum(m, axis=0)[:, None]

    @pl.when(i == NBLK - 1)
    def _():
        pooled = pacc[...] / jnp.maximum(cacc[...], 1.0)
        out_ref[...] = lax.dot(
            pooled, wlin[...], preferred_element_type=jnp.float32) + blin[...]


def _tc_pool(hp, batchp, w_lin, b_lin):
    hp_spec = pl.BlockSpec((PB, D), lambda i: (i, 0))
    return pl.pallas_call(
        _pool_body,
        grid=(NBLK,),
        in_specs=[hp_spec] * 4 + [
            pl.BlockSpec((PB, 4), lambda i: (i, 0)),
            pl.BlockSpec((D, D), lambda i: (0, 0)),
            pl.BlockSpec((1, D), lambda i: (0, 0)),
        ],
        out_specs=pl.BlockSpec((G, D), lambda i: (0, 0)),
        out_shape=jax.ShapeDtypeStruct((G, D), jnp.float32),
        scratch_shapes=[pltpu.VMEM((G, D), jnp.float32),
                        pltpu.VMEM((G, D), jnp.float32)],
    )(*hp, batchp, w_lin, b_lin.reshape(1, D))


def kernel(x, edge_attr, W_rel, W_root, b_conv, W_lin, b_lin, edge_index, batch):
    src = edge_index[0].astype(jnp.int32)
    dst = edge_index[1].astype(jnp.int32)
    edge_type = jnp.argmax(edge_attr, axis=1).astype(jnp.int32)

    keys = edge_type * Np + dst
    pad = EP - E
    ar = jnp.arange(pad, dtype=jnp.int32)
    keys = jnp.concatenate([keys, (ar % R) * Np + N + (ar % (Np - N))])
    srcp = jnp.concatenate([src, ar % N])
    keys3 = keys.reshape(NT, NW, WIN)
    src3 = srcp.reshape(NT, NW, WIN)

    xp = jnp.concatenate([x, jnp.zeros((Np - N, D), x.dtype)], axis=0)
    hp = tuple(
        xp[:, c * CH:(c + 1) * CH].reshape(Np // 4, D) for c in range(4))

    batchp = jnp.concatenate(
        [batch.astype(jnp.int32), jnp.full((Np - N,), G, jnp.int32)]
    ).reshape(Np // 4, 4)

    cntp = None
    for l in range(W_rel.shape[0]):
        hc = tuple(h.reshape(Np, CH) for h in hp)
        if l == 0:
            *a_chunks, cnt = _sc_first(*hc, keys3, src3)
            cntp = cnt.reshape(R, Np).T.reshape(Np // 4, 4 * R)
        else:
            a_chunks = _sc_rest(*hc, keys3, src3)
        ap = tuple(a.reshape(RNp // 4, D) for a in a_chunks)
        hp = _tc_layer(hp, ap, cntp, W_root[l], W_rel[l], b_conv[l])

    return _tc_pool(hp, batchp, W_lin, b_lin)


# speedup vs baseline: 17.1518x; 1.0897x over previous
import functools

import jax
import jax.numpy as jnp
from jax import lax
from jax.experimental import pallas as pl
from jax.experimental.pallas import tpu as pltpu
from jax.experimental.pallas import tpu_sc as plsc

N = 10000
E = 320000
D = 128
R = 4
G = 16
Np = 10240
CH = 32
NT = 16
WIN = 128
GW = 32
NW = 160
NSEG = NW // GW
EPT = NW * WIN
EP = NT * EPT
RNp = R * Np
RPT = RNp // NT
BLK = 1024
NBLK = Np // BLK


def _sc_body(with_cnt, hc0, hc1, hc2, hc3, keys3, src3, *rest):
    if with_cnt:
        a0, a1, a2, a3, cnt = rest[:5]
        scratch = rest[5:]
    else:
        a0, a1, a2, a3 = rest[:4]
        cnt = None
        scratch = rest[4:]
    (acc, cntacc, keys_v, src_v, rows0, rows1, rows2, rows3,
     zb, zr, ones_v, g0, g1, g2, g3, s0, s1, s2, s3,
     c0, c1, c2, c3, osem) = scratch
    rows = (rows0, rows1, rows2, rows3)
    gsem = (g0, g1, g2, g3)
    ssem = (s0, s1, s2, s3)
    csem = (c0, c1, c2, c3)

    core = lax.axis_index("c")
    sid = lax.axis_index("s")

    for j in range(WIN):
        zb[j, pl.ds(0, 16)] = jnp.zeros((16,), jnp.float32)
        zb[j, pl.ds(16, 16)] = jnp.zeros((16,), jnp.float32)
    for j in range(WIN // 16):
        zr[pl.ds(j * 16, 16)] = jnp.zeros((16,), jnp.float32)
        ones_v[pl.ds(j * 16, 16)] = jnp.ones((16,), jnp.float32)

    def zero_acc(also_cnt):
        base = sid * RPT
        for w in range(RPT // WIN):
            pltpu.async_copy(zb, acc.at[pl.ds(base + w * WIN, WIN)], osem)
            if also_cnt:
                pltpu.async_copy(
                    zr, cntacc.at[pl.ds(base + w * WIN, WIN)], osem)
        for w in range(RPT // WIN):
            pltpu.make_async_copy(
                zb, acc.at[pl.ds(base + w * WIN, WIN)], osem).wait()
            if also_cnt:
                pltpu.make_async_copy(
                    zr, cntacc.at[pl.ds(base + w * WIN, WIN)], osem).wait()
        plsc.subcore_barrier()

    def do_pass(table, a_out, count):
        def wait_dma(buf, sem):
            pltpu.make_async_copy(table.at[src_v.at[0]], buf, sem).wait()

        def sbody(s, carry):
            pltpu.sync_copy(keys3.at[sid, pl.ds(s * GW, GW)], keys_v)
            pltpu.sync_copy(src3.at[sid, pl.ds(s * GW, GW)], src_v)
            pltpu.async_copy(table.at[src_v.at[0]], rows0, g0)
            pltpu.async_copy(table.at[src_v.at[1]], rows1, g1)

            def qbody(t, c2):
                for k in range(4):
                    w = 4 * t + k
                    b = k
                    b2 = (k + 2) % 4
                    wait_dma(rows[b], gsem[b])
                    pltpu.async_copy(rows[b], acc.at[keys_v.at[w]],
                                     ssem[b], add=True)
                    if count:
                        @pl.when((s > 0) | (t > 0))
                        def _():
                            pltpu.make_async_copy(
                                cnt.at[pl.ds(0, WIN)], ones_v, csem[b]).wait()
                        pltpu.async_copy(ones_v, cntacc.at[keys_v.at[w]],
                                         csem[b], add=True)
                    if k < 2:
                        @pl.when(t > 0)
                        def _():
                            wait_dma(rows[b2], ssem[b2])
                        pltpu.async_copy(table.at[src_v.at[w + 2]],
                                         rows[b2], gsem[b2])
                    else:
                        wait_dma(rows[b2], ssem[b2])

                        @pl.when(t < GW // 4 - 1)
                        def _():
                            pltpu.async_copy(table.at[src_v.at[w + 2]],
                                             rows[b2], gsem[b2])
                return c2
            lax.fori_loop(0, GW // 4, qbody, 0)
            wait_dma(rows2, s2)
            wait_dma(rows3, s3)
            return carry
        lax.fori_loop(0, NSEG, sbody, 0)
        if count:
            for b in range(4):
                pltpu.make_async_copy(
                    cnt.at[pl.ds(0, WIN)], ones_v, csem[b]).wait()
        plsc.subcore_barrier()
        base = sid * RPT
        for w in range(RPT // WIN):
            off = base + w * WIN
            pltpu.async_copy(
                acc.at[pl.ds(off, WIN)], a_out.at[pl.ds(off, WIN)], osem)
            if count:
                pltpu.async_copy(
                    cntacc.at[pl.ds(off, WIN)], cnt.at[pl.ds(off, WIN)], osem)
        for w in range(RPT // WIN):
            off = base + w * WIN
            pltpu.make_async_copy(
                acc.at[pl.ds(off, WIN)], a_out.at[pl.ds(off, WIN)], osem).wait()
            if count:
                pltpu.make_async_copy(
                    cntacc.at[pl.ds(off, WIN)], cnt.at[pl.ds(off, WIN)],
                    osem).wait()
        plsc.subcore_barrier()

    @pl.when(core == 0)
    def _():
        zero_acc(with_cnt)
        do_pass(hc0, a0, with_cnt)
        zero_acc(False)
        do_pass(hc1, a1, False)

    @pl.when(core == 1)
    def _():
        zero_acc(False)
        do_pass(hc2, a2, False)
        zero_acc(False)
        do_pass(hc3, a3, False)


def _make_sc(with_cnt):
    out_type = [jax.ShapeDtypeStruct((RNp, CH), jnp.float32) for _ in range(4)]
    if with_cnt:
        out_type.append(jax.ShapeDtypeStruct((RNp,), jnp.float32))
    return pl.kernel(
        functools.partial(_sc_body, with_cnt),
        out_type=tuple(out_type),
        mesh=plsc.VectorSubcoreMesh(core_axis_name="c", subcore_axis_name="s"),
        scratch_types=(
            pltpu.VMEM_SHARED((RNp, CH), jnp.float32),
            pltpu.VMEM_SHARED((RNp,), jnp.float32),
            pltpu.VMEM((GW, WIN), jnp.int32),
            pltpu.VMEM((GW, WIN), jnp.int32),
            pltpu.VMEM((WIN, CH), jnp.float32),
            pltpu.VMEM((WIN, CH), jnp.float32),
            pltpu.VMEM((WIN, CH), jnp.float32),
            pltpu.VMEM((WIN, CH), jnp.float32),
            pltpu.VMEM((WIN, CH), jnp.float32),
            pltpu.VMEM((WIN,), jnp.float32),
            pltpu.VMEM((WIN,), jnp.float32),
            pltpu.SemaphoreType.DMA,
            pltpu.SemaphoreType.DMA,
            pltpu.SemaphoreType.DMA,
            pltpu.SemaphoreType.DMA,
            pltpu.SemaphoreType.DMA,
            pltpu.SemaphoreType.DMA,
            pltpu.SemaphoreType.DMA,
            pltpu.SemaphoreType.DMA,
            pltpu.SemaphoreType.DMA,
            pltpu.SemaphoreType.DMA,
            pltpu.SemaphoreType.DMA,
            pltpu.SemaphoreType.DMA,
            pltpu.SemaphoreType.DMA,
        ),
        compiler_params=pltpu.CompilerParams(use_tc_tiling_on_sc=False),
    )


_sc_first = _make_sc(True)
_sc_rest = _make_sc(False)


PB = BLK // 4


def _layer_body(hp0, hp1, hp2, hp3, *refs):
    a = refs[:16]
    cntp, wroot, wrel, b_ref, o0, o1, o2, o3 = refs[16:]
    hp = (hp0, hp1, hp2, hp3)
    invp = 1.0 / jnp.maximum(cntp[...], 1.0)
    ys = []
    for k in range(4):
        hk = jnp.concatenate(
            [hp[c][:, 32 * k:32 * k + 32] for c in range(4)], axis=1)
        yk = lax.dot(hk, wroot[...], preferred_element_type=jnp.float32)
        yk = yk + b_ref[...]
        for r in range(R):
            zrk = jnp.concatenate(
                [a[4 * r + c][:, 32 * k:32 * k + 32] for c in range(4)],
                axis=1)
            yk = yk + lax.dot(
                zrk, wrel[r], preferred_element_type=jnp.float32
            ) * invp[:, 4 * k + r:4 * k + r + 1]
        ys.append(jnp.maximum(yk, 0.0))
    outs = (o0, o1, o2, o3)
    for c in range(4):
        outs[c][...] = jnp.concatenate(
            [ys[k][:, 32 * c:32 * c + 32] for k in range(4)], axis=1)


def _a_spec(r):
    return pl.BlockSpec((PB, D), lambda i, r=r: (r * NBLK + i, 0))


def _tc_layer(hp, ap, cntp, w_root, w_rel, b):
    hp_spec = pl.BlockSpec((PB, D), lambda i: (i, 0))
    a_args = [ap[c] for r in range(R) for c in range(4)]
    a_specs = [_a_spec(r) for r in range(R) for c in range(4)]
    return pl.pallas_call(
        _layer_body,
        grid=(NBLK,),
        in_specs=[hp_spec] * 4 + a_specs + [
            pl.BlockSpec((PB, 4 * R), lambda i: (i, 0)),
            pl.BlockSpec((D, D), lambda i: (0, 0)),
            pl.BlockSpec((R, D, D), lambda i: (0, 0, 0)),
            pl.BlockSpec((1, D), lambda i: (0, 0)),
        ],
        out_specs=[pl.BlockSpec((PB, D), lambda i: (i, 0))] * 4,
        out_shape=[jax.ShapeDtypeStruct((Np // 4, D), jnp.float32)] * 4,
    )(*hp, *a_args, cntp, w_root, w_rel, b.reshape(1, D))


def _pool_body(hp0, hp1, hp2, hp3, batch_ref, wlin, blin, out_ref, pacc, cacc):
    i = pl.program_id(0)
    hp = (hp0, hp1, hp2, hp3)

    @pl.when(i == 0)
    def _():
        pacc[...] = jnp.zeros_like(pacc)
        cacc[...] = jnp.zeros_like(cacc)

    bt = batch_ref[...]
    for k in range(4):
        hk = jnp.concatenate(
            [hp[c][:, 32 * k:32 * k + 32] for c in range(4)], axis=1)
        m = (bt[:, k:k + 1] == lax.broadcasted_iota(jnp.int32, (PB, G), 1))
        m = m.astype(jnp.float32)
        pacc[...] = pacc[...] + lax.dot_general(
            m, hk, (((0,), (0,)), ((), ())),
            preferred_element_type=jnp.float32)
        cacc[...] = cacc[...] + jnp.sum(m, axis=0)[:, None]

    @pl.when(i == NBLK - 1)
    def _():
        pooled = pacc[...] / jnp.maximum(cacc[...], 1.0)
        out_ref[...] = lax.dot(
            pooled, wlin[...], preferred_element_type=jnp.float32) + blin[...]


def _tc_pool(hp, batchp, w_lin, b_lin):
    hp_spec = pl.BlockSpec((PB, D), lambda i: (i, 0))
    return pl.pallas_call(
        _pool_body,
        grid=(NBLK,),
        in_specs=[hp_spec] * 4 + [
            pl.BlockSpec((PB, 4), lambda i: (i, 0)),
            pl.BlockSpec((D, D), lambda i: (0, 0)),
            pl.BlockSpec((1, D), lambda i: (0, 0)),
        ],
        out_specs=pl.BlockSpec((G, D), lambda i: (0, 0)),
        out_shape=jax.ShapeDtypeStruct((G, D), jnp.float32),
        scratch_shapes=[pltpu.VMEM((G, D), jnp.float32),
                        pltpu.VMEM((G, D), jnp.float32)],
    )(*hp, batchp, w_lin, b_lin.reshape(1, D))


def kernel(x, edge_attr, W_rel, W_root, b_conv, W_lin, b_lin, edge_index, batch):
    src = edge_index[0].astype(jnp.int32)
    dst = edge_index[1].astype(jnp.int32)
    edge_type = jnp.argmax(edge_attr, axis=1).astype(jnp.int32)

    keys = edge_type * Np + dst
    pad = EP - E
    ar = jnp.arange(pad, dtype=jnp.int32)
    keys = jnp.concatenate([keys, (ar % R) * Np + N + (ar % (Np - N))])
    srcp = jnp.concatenate([src, ar % N])
    keys3 = keys.reshape(NT, NW, WIN)
    src3 = srcp.reshape(NT, NW, WIN)

    xp = jnp.concatenate([x, jnp.zeros((Np - N, D), x.dtype)], axis=0)
    hp = tuple(
        xp[:, c * CH:(c + 1) * CH].reshape(Np // 4, D) for c in range(4))

    batchp = jnp.concatenate(
        [batch.astype(jnp.int32), jnp.full((Np - N,), G, jnp.int32)]
    ).reshape(Np // 4, 4)

    cntp = None
    for l in range(W_rel.shape[0]):
        hc = tuple(h.reshape(Np, CH) for h in hp)
        if l == 0:
            *a_chunks, cnt = _sc_first(*hc, keys3, src3)
            cntp = cnt.reshape(R, Np).T.reshape(Np // 4, 4 * R)
        else:
            a_chunks = _sc_rest(*hc, keys3, src3)
        ap = tuple(a.reshape(RNp // 4, D) for a in a_chunks)
        hp = _tc_layer(hp, ap, cntp, W_root[l], W_rel[l], b_conv[l])

    return _tc_pool(hp, batchp, W_lin, b_lin)
